# baseline (speedup 1.0000x reference)
import functools

import jax
import jax.numpy as jnp
from jax import lax
from jax.experimental import pallas as pl
from jax.experimental.pallas import tpu as pltpu
from jax.experimental.pallas import tpu_sc as plsc

N = 10000
E = 320000
D = 128
G = 256

NC = 2
NS = 16
CH = 128
NCHUNK = 80
NBUF = 2
HALF = NCHUNK // 2
EPT = NCHUNK * CH
EPAD = NC * NS * EPT
NACC = N + 8
RPT = 624


def _sc_body(x_hbm, src_hbm, dst_hbm, agg_hbm, idx_s, idx_d, *rest):
    rows = rest[:NBUF]
    gsem = rest[NBUF:2 * NBUF]
    ssem = rest[2 * NBUF:3 * NBUF]
    acc = rest[3 * NBUF]
    c = lax.axis_index("c")
    s = lax.axis_index("s")

    def _zfill(i, carry):
        for b in range(NBUF):
            for j in range(D // 16):
                rows[b][i, pl.ds(j * 16, 16)] = jnp.zeros((16,), jnp.float32)
        return carry
    lax.fori_loop(0, CH, _zfill, 0)
    row0 = s * RPT
    for t in range(RPT // CH):
        pltpu.sync_copy(rows[t % NBUF], acc.at[pl.ds(row0 + t * CH, CH)])
    pltpu.sync_copy(rows[0].at[pl.ds(0, RPT - (RPT // CH) * CH)],
                    acc.at[pl.ds(row0 + (RPT // CH) * CH,
                                 RPT - (RPT // CH) * CH)])

    @pl.when(s == NS - 1)
    def _():
        left = NACC - NS * RPT
        for t in range(left // CH):
            pltpu.sync_copy(rows[t % NBUF],
                            acc.at[pl.ds(NS * RPT + t * CH, CH)])
        pltpu.sync_copy(rows[0].at[pl.ds(0, left - (left // CH) * CH)],
                        acc.at[pl.ds(NS * RPT + (left // CH) * CH,
                                     left - (left // CH) * CH)])
    plsc.subcore_barrier()

    wid = c * NS + s

    def _g(j, b):
        return pltpu.make_async_copy(x_hbm.at[idx_s.at[j]], rows[b], gsem[b])

    def _s(j, b):
        return pltpu.make_async_copy(rows[b], acc.at[idx_d.at[j]], ssem[b])

    for h in range(2):
        row_h = wid * NCHUNK + h * HALF
        pltpu.sync_copy(src_hbm.at[pl.ds(row_h, HALF)], idx_s)
        pltpu.sync_copy(dst_hbm.at[pl.ds(row_h, HALF)], idx_d)

        for b in range(NBUF):
            _g(b, b).start()

        def _group(k, carry):
            j0 = k * NBUF
            for b in range(NBUF):
                _g(j0 + b, b).wait()
                _s(j0 + b, b).start(add=True)
            for b in range(NBUF):
                _s(j0 + b, b).wait()
                _g(j0 + NBUF + b, b).start()
            return carry
        lax.fori_loop(0, HALF // NBUF - 1, _group, 0)

        j0 = HALF - NBUF
        for b in range(NBUF):
            _g(j0 + b, b).wait()
            _s(j0 + b, b).start(add=True)
        for b in range(NBUF):
            _s(j0 + b, b).wait()

    plsc.subcore_barrier()

    pltpu.sync_copy(acc.at[pl.ds(row0, RPT)], agg_hbm.at[c, pl.ds(row0, RPT)])

    @pl.when(s == NS - 1)
    def _():
        left = N - NS * RPT
        pltpu.sync_copy(acc.at[pl.ds(NS * RPT, left)],
                        agg_hbm.at[c, pl.ds(NS * RPT, left)])


@jax.jit
def _sc_scatter(x, src2, dst2):
    mesh = plsc.VectorSubcoreMesh(core_axis_name="c", subcore_axis_name="s")
    scratch = [
        pltpu.VMEM((HALF, CH), jnp.int32),
        pltpu.VMEM((HALF, CH), jnp.int32),
    ]
    scratch += [pltpu.VMEM((CH, D), jnp.float32) for _ in range(NBUF)]
    scratch += [pltpu.SemaphoreType.DMA for _ in range(2 * NBUF)]
    scratch += [pltpu.MemorySpace.VMEM_SHARED((NACC, D), jnp.float32)]
    return pl.kernel(
        _sc_body,
        out_type=jax.ShapeDtypeStruct((NC, N, D), jnp.float32),
        mesh=mesh,
        scratch_types=scratch,
    )(x, src2, dst2)


BLK = 400
NBLK = N // BLK


def _tc_body(agg_ref, x_ref, batch_ref, wn_ref, wr_ref, b_ref, out_ref):
    i = pl.program_id(0)
    a = agg_ref[0] + agg_ref[1]
    h = jnp.dot(a, wn_ref[...], preferred_element_type=jnp.float32)
    h = h + jnp.dot(x_ref[...], wr_ref[...], preferred_element_type=jnp.float32)
    h = jnp.maximum(h + b_ref[...], 0.0)
    bt = batch_ref[0, 0, :]
    gid = lax.broadcasted_iota(jnp.int32, (G, BLK), 0)
    onehot = jnp.where(gid == bt[None, :], 1.0, 0.0)
    p = jnp.dot(onehot, h, preferred_element_type=jnp.float32)

    @pl.when(i == 0)
    def _():
        out_ref[...] = p

    @pl.when(i > 0)
    def _():
        out_ref[...] += p


@jax.jit
def _tc_combine(agg2, x, batch3, W_nbr, W_root, b2):
    return pl.pallas_call(
        _tc_body,
        grid=(NBLK,),
        in_specs=[
            pl.BlockSpec((NC, BLK, D), lambda i: (0, i, 0)),
            pl.BlockSpec((BLK, D), lambda i: (i, 0)),
            pl.BlockSpec((1, 1, BLK), lambda i: (i, 0, 0)),
            pl.BlockSpec((D, D), lambda i: (0, 0)),
            pl.BlockSpec((D, D), lambda i: (0, 0)),
            pl.BlockSpec((1, D), lambda i: (0, 0)),
        ],
        out_specs=pl.BlockSpec((G, D), lambda i: (0, 0)),
        out_shape=jax.ShapeDtypeStruct((G, D), jnp.float32),
    )(agg2, x, batch3, W_nbr, W_root, b2)


def kernel(x, edge_index, batch, W_nbr, W_root, b):
    src = edge_index[0]
    dst = edge_index[1]
    pad = EPAD - E
    src2 = jnp.concatenate([src, jnp.zeros((pad,), jnp.int32)]).reshape(-1, CH)
    dst2 = jnp.concatenate([dst, jnp.full((pad,), N, jnp.int32)]).reshape(-1, CH)
    agg2 = _sc_scatter(x, src2, dst2)
    batch3 = batch.reshape(NBLK, 1, BLK)
    b2 = b.reshape(1, D)
    return _tc_combine(agg2, x, batch3, W_nbr, W_root, b2)


# speedup vs baseline: 2.2620x; 2.2620x over previous
import functools

import jax
import jax.numpy as jnp
from jax import lax
from jax.experimental import pallas as pl
from jax.experimental.pallas import tpu as pltpu
from jax.experimental.pallas import tpu_sc as plsc

N = 10000
E = 320000
D = 128
G = 256

NC = 2
NS = 16
CH = 125
NCHUNK = 80
NBUF = 2
HALF = NCHUNK // 2
EPT = NCHUNK * CH
NACC = N
RPT = 624


def _sc_body(x_hbm, src_hbm, dst_hbm, agg_hbm, idx_s, idx_d, *rest):
    rows = rest[:NBUF]
    gsem = rest[NBUF:2 * NBUF]
    ssem = rest[2 * NBUF:3 * NBUF]
    acc = rest[3 * NBUF]
    c = lax.axis_index("c")
    s = lax.axis_index("s")

    def _zfill(i, carry):
        for b in range(NBUF):
            for j in range(D // 16):
                rows[b][i, pl.ds(j * 16, 16)] = jnp.zeros((16,), jnp.float32)
        return carry
    lax.fori_loop(0, CH, _zfill, 0)
    row0 = s * RPT
    ZC = 104
    for t in range(RPT // ZC):
        pltpu.sync_copy(rows[t % NBUF].at[pl.ds(0, ZC)],
                        acc.at[pl.ds(row0 + t * ZC, ZC)])

    @pl.when(s == NS - 1)
    def _():
        left = NACC - NS * RPT
        pltpu.sync_copy(rows[0].at[pl.ds(0, left)],
                        acc.at[pl.ds(NS * RPT, left)])
    plsc.subcore_barrier()

    wid = c * NS + s

    def _g(j, b):
        return pltpu.make_async_copy(x_hbm.at[idx_s.at[j]], rows[b], gsem[b])

    def _s(j, b):
        return pltpu.make_async_copy(rows[b], acc.at[idx_d.at[j]], ssem[b])

    for h in range(2):
        row_h = wid * NCHUNK + h * HALF
        pltpu.sync_copy(src_hbm.at[pl.ds(row_h, HALF)], idx_s)
        pltpu.sync_copy(dst_hbm.at[pl.ds(row_h, HALF)], idx_d)

        for b in range(NBUF):
            _g(b, b).start()

        def _group(k, carry):
            j0 = k * NBUF
            for b in range(NBUF):
                _g(j0 + b, b).wait()
                _s(j0 + b, b).start(add=True)
            for b in range(NBUF):
                _s(j0 + b, b).wait()
                _g(j0 + NBUF + b, b).start()
            return carry
        lax.fori_loop(0, HALF // NBUF - 1, _group, 0)

        j0 = HALF - NBUF
        for b in range(NBUF):
            _g(j0 + b, b).wait()
            _s(j0 + b, b).start(add=True)
        for b in range(NBUF):
            _s(j0 + b, b).wait()

    plsc.subcore_barrier()

    pltpu.sync_copy(acc.at[pl.ds(row0, RPT)], agg_hbm.at[c, pl.ds(row0, RPT)])

    @pl.when(s == NS - 1)
    def _():
        left = N - NS * RPT
        pltpu.sync_copy(acc.at[pl.ds(NS * RPT, left)],
                        agg_hbm.at[c, pl.ds(NS * RPT, left)])


@jax.jit
def _sc_scatter(x, src2, dst2):
    mesh = plsc.VectorSubcoreMesh(core_axis_name="c", subcore_axis_name="s")
    scratch = [
        pltpu.VMEM((HALF, CH), jnp.int32),
        pltpu.VMEM((HALF, CH), jnp.int32),
    ]
    scratch += [pltpu.VMEM((CH, D), jnp.float32) for _ in range(NBUF)]
    scratch += [pltpu.SemaphoreType.DMA for _ in range(2 * NBUF)]
    scratch += [pltpu.MemorySpace.VMEM_SHARED((NACC, D), jnp.float32)]
    return pl.kernel(
        _sc_body,
        out_type=jax.ShapeDtypeStruct((NC, N, D), jnp.float32),
        mesh=mesh,
        scratch_types=scratch,
    )(x, src2, dst2)


BLK = 400
NBLK = N // BLK


def _tc_body(agg_ref, x_ref, batch_ref, wn_ref, wr_ref, b_ref, out_ref):
    i = pl.program_id(0)
    a = agg_ref[0] + agg_ref[1]
    h = jnp.dot(a, wn_ref[...], preferred_element_type=jnp.float32)
    h = h + jnp.dot(x_ref[...], wr_ref[...], preferred_element_type=jnp.float32)
    h = jnp.maximum(h + b_ref[...], 0.0)
    bt = batch_ref[0, 0, :]
    gid = lax.broadcasted_iota(jnp.int32, (G, BLK), 0)
    onehot = jnp.where(gid == bt[None, :], 1.0, 0.0)
    p = jnp.dot(onehot, h, preferred_element_type=jnp.float32)

    @pl.when(i == 0)
    def _():
        out_ref[...] = p

    @pl.when(i > 0)
    def _():
        out_ref[...] += p


@jax.jit
def _tc_combine(agg2, x, batch3, W_nbr, W_root, b2):
    return pl.pallas_call(
        _tc_body,
        grid=(NBLK,),
        in_specs=[
            pl.BlockSpec((NC, BLK, D), lambda i: (0, i, 0)),
            pl.BlockSpec((BLK, D), lambda i: (i, 0)),
            pl.BlockSpec((1, 1, BLK), lambda i: (i, 0, 0)),
            pl.BlockSpec((D, D), lambda i: (0, 0)),
            pl.BlockSpec((D, D), lambda i: (0, 0)),
            pl.BlockSpec((1, D), lambda i: (0, 0)),
        ],
        out_specs=pl.BlockSpec((G, D), lambda i: (0, 0)),
        out_shape=jax.ShapeDtypeStruct((G, D), jnp.float32),
    )(agg2, x, batch3, W_nbr, W_root, b2)


def kernel(x, edge_index, batch, W_nbr, W_root, b):
    src = edge_index[0]
    dst = edge_index[1]
    src2 = src.reshape(-1, CH)
    dst2 = dst.reshape(-1, CH)
    agg2 = _sc_scatter(x, src2, dst2)
    batch3 = batch.reshape(NBLK, 1, BLK)
    b2 = b.reshape(1, D)
    return _tc_combine(agg2, x, batch3, W_nbr, W_root, b2)


# speedup vs baseline: 2.3564x; 1.0417x over previous
import functools

import jax
import jax.numpy as jnp
from jax import lax
from jax.experimental import pallas as pl
from jax.experimental.pallas import tpu as pltpu
from jax.experimental.pallas import tpu_sc as plsc

N = 10000
E = 320000
D = 128
G = 256

NC = 2
NS = 16
DH = D // NC
CH = 125
NCHUNK = E // (NS * CH)
NBUF = 4
RPT = 624


def _sc_body(x2_hbm, src_hbm, dst_hbm, agg_hbm, idx_s, idx_d, *rest):
    rows = rest[:NBUF]
    gsem = rest[NBUF:2 * NBUF]
    ssem = rest[2 * NBUF:3 * NBUF]
    acc = rest[3 * NBUF]
    c = lax.axis_index("c")
    s = lax.axis_index("s")

    def _zfill(i, carry):
        for b in range(NBUF):
            for j in range(DH // 16):
                rows[b][i, pl.ds(j * 16, 16)] = jnp.zeros((16,), jnp.float32)
        return carry
    lax.fori_loop(0, CH, _zfill, 0)
    row0 = s * RPT
    ZC = 104
    for t in range(RPT // ZC):
        pltpu.sync_copy(rows[t % NBUF].at[pl.ds(0, ZC)],
                        acc.at[pl.ds(row0 + t * ZC, ZC)])

    @pl.when(s == NS - 1)
    def _():
        left = N - NS * RPT
        pltpu.sync_copy(rows[0].at[pl.ds(0, left)],
                        acc.at[pl.ds(NS * RPT, left)])
    plsc.subcore_barrier()

    pltpu.sync_copy(src_hbm.at[pl.ds(s * NCHUNK, NCHUNK)], idx_s)
    pltpu.sync_copy(dst_hbm.at[pl.ds(s * NCHUNK, NCHUNK)], idx_d)

    def _g(j, b):
        return pltpu.make_async_copy(x2_hbm.at[c].at[idx_s.at[j]], rows[b],
                                     gsem[b])

    def _s(j, b):
        return pltpu.make_async_copy(rows[b], acc.at[idx_d.at[j]], ssem[b])

    for b in range(NBUF):
        _g(b, b).start()

    def _group(k, carry):
        j0 = k * NBUF
        for b in range(NBUF):
            _g(j0 + b, b).wait()
            _s(j0 + b, b).start(add=True)
        for b in range(NBUF):
            _s(j0 + b, b).wait()
            _g(j0 + NBUF + b, b).start()
        return carry
    lax.fori_loop(0, NCHUNK // NBUF - 1, _group, 0)

    j0 = NCHUNK - NBUF
    for b in range(NBUF):
        _g(j0 + b, b).wait()
        _s(j0 + b, b).start(add=True)
    for b in range(NBUF):
        _s(j0 + b, b).wait()

    plsc.subcore_barrier()

    pltpu.sync_copy(acc.at[pl.ds(row0, RPT)], agg_hbm.at[c, pl.ds(row0, RPT)])

    @pl.when(s == NS - 1)
    def _():
        left = N - NS * RPT
        pltpu.sync_copy(acc.at[pl.ds(NS * RPT, left)],
                        agg_hbm.at[c, pl.ds(NS * RPT, left)])


@jax.jit
def _sc_scatter(x2, src2, dst2):
    mesh = plsc.VectorSubcoreMesh(core_axis_name="c", subcore_axis_name="s")
    scratch = [
        pltpu.VMEM((NCHUNK, CH), jnp.int32),
        pltpu.VMEM((NCHUNK, CH), jnp.int32),
    ]
    scratch += [pltpu.VMEM((CH, DH), jnp.float32) for _ in range(NBUF)]
    scratch += [pltpu.SemaphoreType.DMA for _ in range(2 * NBUF)]
    scratch += [pltpu.MemorySpace.VMEM_SHARED((N, DH), jnp.float32)]
    return pl.kernel(
        _sc_body,
        out_type=jax.ShapeDtypeStruct((NC, N, DH), jnp.float32),
        mesh=mesh,
        scratch_types=scratch,
        compiler_params=pltpu.CompilerParams(use_tc_tiling_on_sc=False),
    )(x2, src2, dst2)


BLK = 400
NBLK = N // BLK


def _tc_body(agg_ref, x_ref, batch_ref, wn_ref, wr_ref, b_ref, out_ref):
    i = pl.program_id(0)
    a = jnp.concatenate([agg_ref[0], agg_ref[1]], axis=1)
    h = jnp.dot(a, wn_ref[...], preferred_element_type=jnp.float32)
    h = h + jnp.dot(x_ref[...], wr_ref[...], preferred_element_type=jnp.float32)
    h = jnp.maximum(h + b_ref[...], 0.0)
    bt = batch_ref[0, 0, :]
    gid = lax.broadcasted_iota(jnp.int32, (G, BLK), 0)
    onehot = jnp.where(gid == bt[None, :], 1.0, 0.0)
    p = jnp.dot(onehot, h, preferred_element_type=jnp.float32)

    @pl.when(i == 0)
    def _():
        out_ref[...] = p

    @pl.when(i > 0)
    def _():
        out_ref[...] += p


@jax.jit
def _tc_combine(agg2, x, batch3, W_nbr, W_root, b2):
    return pl.pallas_call(
        _tc_body,
        grid=(NBLK,),
        in_specs=[
            pl.BlockSpec((NC, BLK, DH), lambda i: (0, i, 0)),
            pl.BlockSpec((BLK, D), lambda i: (i, 0)),
            pl.BlockSpec((1, 1, BLK), lambda i: (i, 0, 0)),
            pl.BlockSpec((D, D), lambda i: (0, 0)),
            pl.BlockSpec((D, D), lambda i: (0, 0)),
            pl.BlockSpec((1, D), lambda i: (0, 0)),
        ],
        out_specs=pl.BlockSpec((G, D), lambda i: (0, 0)),
        out_shape=jax.ShapeDtypeStruct((G, D), jnp.float32),
    )(agg2, x, batch3, W_nbr, W_root, b2)


def kernel(x, edge_index, batch, W_nbr, W_root, b):
    src = edge_index[0]
    dst = edge_index[1]
    src2 = src.reshape(-1, CH)
    dst2 = dst.reshape(-1, CH)
    x2 = x.reshape(N, NC, DH).transpose(1, 0, 2)
    agg2 = _sc_scatter(x2, src2, dst2)
    batch3 = batch.reshape(NBLK, 1, BLK)
    b2 = b.reshape(1, D)
    return _tc_combine(agg2, x, batch3, W_nbr, W_root, b2)


# speedup vs baseline: 2.7609x; 1.1717x over previous
import functools

import jax
import jax.numpy as jnp
from jax import lax
from jax.experimental import pallas as pl
from jax.experimental.pallas import tpu as pltpu
from jax.experimental.pallas import tpu_sc as plsc

N = 10000
E = 320000
D = 128
G = 256

NC = 2
NS = 16
DH = D // NC
CH = 80
NCHUNK = E // (NS * CH)
NROWS = E // CH
NBUF = 5
RPT = 624


def _sc_body(x_hbm, ei_hbm, agg_hbm, idx_s, idx_d, *rest):
    rows = rest[:NBUF]
    gsem = rest[NBUF:2 * NBUF]
    ssem = rest[2 * NBUF:3 * NBUF]
    acc = rest[3 * NBUF]
    c = lax.axis_index("c")
    s = lax.axis_index("s")

    def _zfill(i, carry):
        for b in range(NBUF):
            for j in range(DH // 16):
                rows[b][i, pl.ds(j * 16, 16)] = jnp.zeros((16,), jnp.float32)
        return carry
    lax.fori_loop(0, CH, _zfill, 0)
    row0 = s * RPT
    ZC = 80
    for t in range(RPT // ZC):
        pltpu.sync_copy(rows[t % NBUF], acc.at[pl.ds(row0 + t * ZC, ZC)])
    pltpu.sync_copy(rows[0].at[pl.ds(0, RPT - (RPT // ZC) * ZC)],
                    acc.at[pl.ds(row0 + (RPT // ZC) * ZC,
                                 RPT - (RPT // ZC) * ZC)])

    @pl.when(s == NS - 1)
    def _():
        left = N - NS * RPT
        pltpu.sync_copy(rows[0].at[pl.ds(0, left)],
                        acc.at[pl.ds(NS * RPT, left)])
    plsc.subcore_barrier()

    pltpu.sync_copy(ei_hbm.at[0].at[pl.ds(s * NCHUNK, NCHUNK)], idx_s)
    pltpu.sync_copy(ei_hbm.at[1].at[pl.ds(s * NCHUNK, NCHUNK)], idx_d)

    def _g(j, b):
        return pltpu.make_async_copy(
            x_hbm.at[c].at[idx_s.at[j]], rows[b], gsem[b])

    def _s(j, b):
        return pltpu.make_async_copy(rows[b], acc.at[idx_d.at[j]], ssem[b])

    for b in range(NBUF):
        _g(b, b).start()

    def _group(k, carry):
        j0 = k * NBUF
        for b in range(NBUF):
            _g(j0 + b, b).wait()
            _s(j0 + b, b).start(add=True)
        for b in range(NBUF):
            _s(j0 + b, b).wait()
            _g(j0 + NBUF + b, b).start()
        return carry
    lax.fori_loop(0, NCHUNK // NBUF - 1, _group, 0)

    j0 = NCHUNK - NBUF
    for b in range(NBUF):
        _g(j0 + b, b).wait()
        _s(j0 + b, b).start(add=True)
    for b in range(NBUF):
        _s(j0 + b, b).wait()

    plsc.subcore_barrier()

    pltpu.sync_copy(acc.at[pl.ds(row0, RPT)], agg_hbm.at[c, pl.ds(row0, RPT)])

    @pl.when(s == NS - 1)
    def _():
        left = N - NS * RPT
        pltpu.sync_copy(acc.at[pl.ds(NS * RPT, left)],
                        agg_hbm.at[c, pl.ds(NS * RPT, left)])


@jax.jit
def _sc_scatter(x, ei3):
    mesh = plsc.VectorSubcoreMesh(core_axis_name="c", subcore_axis_name="s")
    scratch = [
        pltpu.VMEM((NCHUNK, CH), jnp.int32),
        pltpu.VMEM((NCHUNK, CH), jnp.int32),
    ]
    scratch += [pltpu.VMEM((CH, DH), jnp.float32) for _ in range(NBUF)]
    scratch += [pltpu.SemaphoreType.DMA for _ in range(2 * NBUF)]
    scratch += [pltpu.MemorySpace.VMEM_SHARED((N, DH), jnp.float32)]
    return pl.kernel(
        _sc_body,
        out_type=jax.ShapeDtypeStruct((NC, N, DH), jnp.float32),
        mesh=mesh,
        scratch_types=scratch,
        compiler_params=pltpu.CompilerParams(use_tc_tiling_on_sc=False),
    )(x, ei3)


BLK = 2000
NBLK = N // BLK


def _tc_body(agg_ref, x_ref, batch_ref, wn_ref, wr_ref, b_ref, out_ref):
    i = pl.program_id(0)
    a = jnp.concatenate([agg_ref[0], agg_ref[1]], axis=1)
    h = jnp.dot(a, wn_ref[...], preferred_element_type=jnp.float32)
    h = h + jnp.dot(x_ref[...], wr_ref[...], preferred_element_type=jnp.float32)
    h = jnp.maximum(h + b_ref[...], 0.0)
    bt = batch_ref[0, 0, :]
    gid = lax.broadcasted_iota(jnp.int32, (G, BLK), 0)
    onehot = jnp.where(gid == bt[None, :], 1.0, 0.0)
    p = jnp.dot(onehot, h, preferred_element_type=jnp.float32)

    @pl.when(i == 0)
    def _():
        out_ref[...] = p

    @pl.when(i > 0)
    def _():
        out_ref[...] += p


@jax.jit
def _tc_combine(agg2, x, batch3, W_nbr, W_root, b2):
    return pl.pallas_call(
        _tc_body,
        grid=(NBLK,),
        in_specs=[
            pl.BlockSpec((NC, BLK, DH), lambda i: (0, i, 0)),
            pl.BlockSpec((BLK, D), lambda i: (i, 0)),
            pl.BlockSpec((1, 1, BLK), lambda i: (i, 0, 0)),
            pl.BlockSpec((D, D), lambda i: (0, 0)),
            pl.BlockSpec((D, D), lambda i: (0, 0)),
            pl.BlockSpec((1, D), lambda i: (0, 0)),
        ],
        out_specs=pl.BlockSpec((G, D), lambda i: (0, 0)),
        out_shape=jax.ShapeDtypeStruct((G, D), jnp.float32),
    )(agg2, x, batch3, W_nbr, W_root, b2)


def kernel(x, edge_index, batch, W_nbr, W_root, b):
    ei3 = edge_index.reshape(2, NROWS, CH)
    x2 = x.reshape(N, NC, DH).transpose(1, 0, 2)
    agg2 = _sc_scatter(x2, ei3)
    batch3 = batch.reshape(NBLK, 1, BLK)
    b2 = b.reshape(1, D)
    return _tc_combine(agg2, x, batch3, W_nbr, W_root, b2)


# speedup vs baseline: 3.0357x; 1.0995x over previous
import functools

import jax
import jax.numpy as jnp
from jax import lax
from jax.experimental import pallas as pl
from jax.experimental.pallas import tpu as pltpu
from jax.experimental.pallas import tpu_sc as plsc

N = 10000
E = 320000
D = 128
G = 256

NC = 2
NS = 16
CH = 128
NCHUNKS = E // CH
COMMON = NCHUNKS // (NC * NS)
EXTRA = NCHUNKS - COMMON * NC * NS
NBUF = 3
RPT = 624


def _sc_body(x_hbm, ei_hbm, agg_hbm, *rest):
    isrc = rest[:2 * NBUF]
    idst = rest[2 * NBUF:4 * NBUF]
    isem = rest[4 * NBUF:6 * NBUF]
    rows = rest[6 * NBUF:7 * NBUF]
    gsem = rest[7 * NBUF:8 * NBUF]
    ssem = rest[8 * NBUF:9 * NBUF]
    acc = rest[9 * NBUF]
    c = lax.axis_index("c")
    s = lax.axis_index("s")
    t = c * NS + s

    def _zfill(i, carry):
        for b in range(NBUF):
            for j in range(D // 16):
                rows[b][i, pl.ds(j * 16, 16)] = jnp.zeros((16,), jnp.float32)
        return carry
    lax.fori_loop(0, CH, _zfill, 0)
    row0 = s * RPT
    for k in range(RPT // CH):
        pltpu.sync_copy(rows[k % NBUF], acc.at[pl.ds(row0 + k * CH, CH)])
    pltpu.sync_copy(rows[0].at[pl.ds(0, RPT - (RPT // CH) * CH)],
                    acc.at[pl.ds(row0 + (RPT // CH) * CH,
                                 RPT - (RPT // CH) * CH)])

    @pl.when(s == NS - 1)
    def _():
        left = N - NS * RPT
        pltpu.sync_copy(rows[0].at[pl.ds(0, left)],
                        acc.at[pl.ds(NS * RPT, left)])
    plsc.subcore_barrier()

    base = t * COMMON

    def _e(j):
        return pl.multiple_of((base + j) * CH, CH)

    def _isrc(j, p, b):
        return pltpu.make_async_copy(ei_hbm.at[0, pl.ds(_e(j), CH)],
                                     isrc[p * NBUF + b], isem[p * NBUF + b])

    def _idst(j, p, b):
        return pltpu.make_async_copy(ei_hbm.at[1, pl.ds(_e(j), CH)],
                                     idst[p * NBUF + b], isem[p * NBUF + b])

    def _g(j, p, b):
        return pltpu.make_async_copy(x_hbm.at[isrc[p * NBUF + b]], rows[b],
                                     gsem[b])

    def _s(j, p, b):
        return pltpu.make_async_copy(rows[b], acc.at[idst[p * NBUF + b]],
                                     ssem[b])

    for b in range(NBUF):
        _isrc(b, 0, b).start()
        _idst(b, 0, b).start()
    for b in range(NBUF):
        _isrc(b, 0, b).wait()
        _idst(b, 0, b).wait()
        _g(b, 0, b).start()

    def _one_group(g, par, prefetch):
        j0 = g * NBUF
        for b in range(NBUF):
            _g(j0 + b, par, b).wait()
            _s(j0 + b, par, b).start(add=True)
            if prefetch:
                _isrc(j0 + NBUF + b, 1 - par, b).start()
                _idst(j0 + NBUF + b, 1 - par, b).start()
        for b in range(NBUF):
            _s(j0 + b, par, b).wait()
            if prefetch:
                _isrc(j0 + NBUF + b, 1 - par, b).wait()
                _idst(j0 + NBUF + b, 1 - par, b).wait()
                _g(j0 + NBUF + b, 1 - par, b).start()

    def _pair(k2, carry):
        _one_group(2 * k2, 0, True)
        _one_group(2 * k2 + 1, 1, True)
        return carry
    NG = COMMON // NBUF
    lax.fori_loop(0, NG // 2 - 1, _pair, 0)
    _one_group(NG - 2, 0, True)
    _one_group(NG - 1, 1, False)

    @pl.when(s == 0)
    def _():
        for q in range(EXTRA // NC):
            off = pl.multiple_of(
                (NC * NS * COMMON + c * (EXTRA // NC) + q) * CH, CH)
            pltpu.sync_copy(ei_hbm.at[0, pl.ds(off, CH)], isrc[0])
            pltpu.sync_copy(ei_hbm.at[1, pl.ds(off, CH)], idst[0])
            _g(0, 0, 0).start()
            _g(0, 0, 0).wait()
            pltpu.sync_copy(rows[0], acc.at[idst[0]], add=True)

    plsc.subcore_barrier()

    pltpu.sync_copy(acc.at[pl.ds(row0, RPT)], agg_hbm.at[c, pl.ds(row0, RPT)])

    @pl.when(s == NS - 1)
    def _():
        left = N - NS * RPT
        pltpu.sync_copy(acc.at[pl.ds(NS * RPT, left)],
                        agg_hbm.at[c, pl.ds(NS * RPT, left)])


@jax.jit
def _sc_scatter(x, edge_index):
    mesh = plsc.VectorSubcoreMesh(core_axis_name="c", subcore_axis_name="s")
    scratch = [pltpu.VMEM((CH,), jnp.int32) for _ in range(4 * NBUF)]
    scratch += [pltpu.SemaphoreType.DMA for _ in range(2 * NBUF)]
    scratch += [pltpu.VMEM((CH, D), jnp.float32) for _ in range(NBUF)]
    scratch += [pltpu.SemaphoreType.DMA for _ in range(2 * NBUF)]
    scratch += [pltpu.MemorySpace.VMEM_SHARED((N, D), jnp.float32)]
    return pl.kernel(
        _sc_body,
        out_type=jax.ShapeDtypeStruct((NC, N, D), jnp.float32),
        mesh=mesh,
        scratch_types=scratch,
    )(x, edge_index)


BLK = 2000
NBLK = N // BLK


def _tc_body(agg_ref, x_ref, batch_ref, wn_ref, wr_ref, b_ref, out_ref):
    i = pl.program_id(0)
    a = agg_ref[0] + agg_ref[1]
    h = jnp.dot(a, wn_ref[...], preferred_element_type=jnp.float32)
    h = h + jnp.dot(x_ref[...], wr_ref[...], preferred_element_type=jnp.float32)
    h = jnp.maximum(h + b_ref[...], 0.0)
    bt = batch_ref[0, 0, :]
    gid = lax.broadcasted_iota(jnp.int32, (G, BLK), 0)
    onehot = jnp.where(gid == bt[None, :], 1.0, 0.0)
    p = jnp.dot(onehot, h, preferred_element_type=jnp.float32)

    @pl.when(i == 0)
    def _():
        out_ref[...] = p

    @pl.when(i > 0)
    def _():
        out_ref[...] += p


@jax.jit
def _tc_combine(agg2, x, batch3, W_nbr, W_root, b2):
    return pl.pallas_call(
        _tc_body,
        grid=(NBLK,),
        in_specs=[
            pl.BlockSpec((NC, BLK, D), lambda i: (0, i, 0)),
            pl.BlockSpec((BLK, D), lambda i: (i, 0)),
            pl.BlockSpec((1, 1, BLK), lambda i: (i, 0, 0)),
            pl.BlockSpec((D, D), lambda i: (0, 0)),
            pl.BlockSpec((D, D), lambda i: (0, 0)),
            pl.BlockSpec((1, D), lambda i: (0, 0)),
        ],
        out_specs=pl.BlockSpec((G, D), lambda i: (0, 0)),
        out_shape=jax.ShapeDtypeStruct((G, D), jnp.float32),
    )(agg2, x, batch3, W_nbr, W_root, b2)


def kernel(x, edge_index, batch, W_nbr, W_root, b):
    agg2 = _sc_scatter(x, edge_index)
    batch3 = batch.reshape(NBLK, 1, BLK)
    b2 = b.reshape(1, D)
    return _tc_combine(agg2, x, batch3, W_nbr, W_root, b2)


# speedup vs baseline: 3.0982x; 1.0206x over previous
import functools

import jax
import jax.numpy as jnp
from jax import lax
from jax.experimental import pallas as pl
from jax.experimental.pallas import tpu as pltpu
from jax.experimental.pallas import tpu_sc as plsc

N = 10000
E = 320000
D = 128
G = 256

NC = 2
NS = 16
CH = 128
NCHUNKS = E // CH
COMMON = NCHUNKS // (NC * NS)
EXTRA = NCHUNKS - COMMON * NC * NS
NBUF = 3
RPT = 624


def _sc_body(x_hbm, ei_hbm, agg_hbm, *rest):
    isrc = rest[:2 * NBUF]
    idst = rest[2 * NBUF:4 * NBUF]
    isem = rest[4 * NBUF:6 * NBUF]
    rows = rest[6 * NBUF:7 * NBUF]
    gsem = rest[7 * NBUF:8 * NBUF]
    ssem = rest[8 * NBUF:9 * NBUF]
    acc = rest[9 * NBUF]
    c = lax.axis_index("c")
    s = lax.axis_index("s")
    t = c * NS + s

    def _zfill(i, carry):
        for b in range(NBUF):
            for j in range(D // 16):
                rows[b][i, pl.ds(j * 16, 16)] = jnp.zeros((16,), jnp.float32)
        return carry
    lax.fori_loop(0, CH, _zfill, 0)
    row0 = s * RPT
    for k in range(RPT // CH):
        pltpu.sync_copy(rows[k % NBUF], acc.at[pl.ds(row0 + k * CH, CH)])
    pltpu.sync_copy(rows[0].at[pl.ds(0, RPT - (RPT // CH) * CH)],
                    acc.at[pl.ds(row0 + (RPT // CH) * CH,
                                 RPT - (RPT // CH) * CH)])

    @pl.when(s == NS - 1)
    def _():
        left = N - NS * RPT
        pltpu.sync_copy(rows[0].at[pl.ds(0, left)],
                        acc.at[pl.ds(NS * RPT, left)])
    plsc.subcore_barrier()

    base = t * COMMON

    def _e(j):
        return pl.multiple_of((base + j) * CH, CH)

    def _isrc(j, p, b):
        return pltpu.make_async_copy(ei_hbm.at[0, pl.ds(_e(j), CH)],
                                     isrc[p * NBUF + b], isem[p * NBUF + b])

    def _idst(j, p, b):
        return pltpu.make_async_copy(ei_hbm.at[1, pl.ds(_e(j), CH)],
                                     idst[p * NBUF + b], isem[p * NBUF + b])

    def _g(j, p, b):
        return pltpu.make_async_copy(x_hbm.at[isrc[p * NBUF + b]], rows[b],
                                     gsem[b])

    def _s(j, p, b):
        return pltpu.make_async_copy(rows[b], acc.at[idst[p * NBUF + b]],
                                     ssem[b])

    for b in range(NBUF):
        _isrc(b, 0, b).start()
        _idst(b, 0, b).start()
    for b in range(NBUF):
        _isrc(b, 0, b).wait()
        _idst(b, 0, b).wait()
        _g(b, 0, b).start()

    def _one_group(g, par, prefetch):
        j0 = g * NBUF
        for b in range(NBUF):
            _g(j0 + b, par, b).wait()
            _s(j0 + b, par, b).start(add=True)
            if prefetch:
                _isrc(j0 + NBUF + b, 1 - par, b).start()
                _idst(j0 + NBUF + b, 1 - par, b).start()
        for b in range(NBUF):
            _s(j0 + b, par, b).wait()
            if prefetch:
                _isrc(j0 + NBUF + b, 1 - par, b).wait()
                _idst(j0 + NBUF + b, 1 - par, b).wait()
                _g(j0 + NBUF + b, 1 - par, b).start()

    def _pair(k2, carry):
        _one_group(2 * k2, 0, True)
        _one_group(2 * k2 + 1, 1, True)
        return carry
    NG = COMMON // NBUF
    lax.fori_loop(0, NG // 2 - 1, _pair, 0)
    _one_group(NG - 2, 0, True)
    _one_group(NG - 1, 1, False)

    @pl.when(s < EXTRA // NC)
    def _():
        off = pl.multiple_of(
            (NC * NS * COMMON) * CH + (c * (EXTRA // NC) + s) * CH, CH)
        pltpu.sync_copy(ei_hbm.at[0, pl.ds(off, CH)], isrc[0])
        pltpu.sync_copy(ei_hbm.at[1, pl.ds(off, CH)], idst[0])
        _g(0, 0, 0).start()
        _g(0, 0, 0).wait()
        pltpu.sync_copy(rows[0], acc.at[idst[0]], add=True)

    plsc.subcore_barrier()

    pltpu.sync_copy(acc.at[pl.ds(row0, RPT)], agg_hbm.at[c, pl.ds(row0, RPT)])

    @pl.when(s == NS - 1)
    def _():
        left = N - NS * RPT
        pltpu.sync_copy(acc.at[pl.ds(NS * RPT, left)],
                        agg_hbm.at[c, pl.ds(NS * RPT, left)])


@jax.jit
def _sc_scatter(x, edge_index):
    mesh = plsc.VectorSubcoreMesh(core_axis_name="c", subcore_axis_name="s")
    scratch = [pltpu.VMEM((CH,), jnp.int32) for _ in range(4 * NBUF)]
    scratch += [pltpu.SemaphoreType.DMA for _ in range(2 * NBUF)]
    scratch += [pltpu.VMEM((CH, D), jnp.float32) for _ in range(NBUF)]
    scratch += [pltpu.SemaphoreType.DMA for _ in range(2 * NBUF)]
    scratch += [pltpu.MemorySpace.VMEM_SHARED((N, D), jnp.float32)]
    return pl.kernel(
        _sc_body,
        out_type=jax.ShapeDtypeStruct((NC, N, D), jnp.float32),
        mesh=mesh,
        scratch_types=scratch,
    )(x, edge_index)


BLK = 2000
NBLK = N // BLK


def _tc_body(agg_ref, x_ref, batch_ref, wn_ref, wr_ref, b_ref, out_ref):
    i = pl.program_id(0)
    a = agg_ref[0] + agg_ref[1]
    h = jnp.dot(a, wn_ref[...], preferred_element_type=jnp.float32)
    h = h + jnp.dot(x_ref[...], wr_ref[...], preferred_element_type=jnp.float32)
    h = jnp.maximum(h + b_ref[...], 0.0)
    bt = batch_ref[0, 0, :]
    gid = lax.broadcasted_iota(jnp.int32, (G, BLK), 0)
    onehot = jnp.where(gid == bt[None, :], 1.0, 0.0)
    p = jnp.dot(onehot, h, preferred_element_type=jnp.float32)

    @pl.when(i == 0)
    def _():
        out_ref[...] = p

    @pl.when(i > 0)
    def _():
        out_ref[...] += p


@jax.jit
def _tc_combine(agg2, x, batch3, W_nbr, W_root, b2):
    return pl.pallas_call(
        _tc_body,
        grid=(NBLK,),
        in_specs=[
            pl.BlockSpec((NC, BLK, D), lambda i: (0, i, 0)),
            pl.BlockSpec((BLK, D), lambda i: (i, 0)),
            pl.BlockSpec((1, 1, BLK), lambda i: (i, 0, 0)),
            pl.BlockSpec((D, D), lambda i: (0, 0)),
            pl.BlockSpec((D, D), lambda i: (0, 0)),
            pl.BlockSpec((1, D), lambda i: (0, 0)),
        ],
        out_specs=pl.BlockSpec((G, D), lambda i: (0, 0)),
        out_shape=jax.ShapeDtypeStruct((G, D), jnp.float32),
    )(agg2, x, batch3, W_nbr, W_root, b2)


def kernel(x, edge_index, batch, W_nbr, W_root, b):
    agg2 = _sc_scatter(x, edge_index)
    batch3 = batch.reshape(NBLK, 1, BLK)
    b2 = b.reshape(1, D)
    return _tc_combine(agg2, x, batch3, W_nbr, W_root, b2)
